# baseline (speedup 1.0000x reference)
import jax
import jax.numpy as jnp
from jax import lax
from jax.experimental import pallas as pl
from jax.experimental.pallas import tpu as pltpu
from jax.experimental.pallas import tpu_sc as plsc

_B, _S, _D = 4, 4096, 1024
_NC, _NS, _L = 2, 16, 16
_NW = _NC * _NS
_ROWS_PER_W = _S // _NW
_C = 16
_NCH = _ROWS_PER_W // _C
_JOBS = [(ci, b) for ci in range(_NCH) for b in range(_B)]
_NJ = len(_JOBS)


def _sc_body(x_hbm, w_hbm, o_hbm,
             wb0, wb1, wb2, tb0, tb1, tb2, sp,
             sw0, sw1, sw2,
             sa0, sa1, sa2, sb0, sb1, sb2, sc0, sc1):
    wbufs = (wb0, wb1, wb2)
    tbufs = (tb0, tb1, tb2)
    sws = (sw0, sw1, sw2)
    s_in = (sa0, sa1, sa2)
    s_back = (sb0, sb1, sb2)
    s_out = (sc0, sc1)

    wid = lax.axis_index("s") * _NC + lax.axis_index("c")
    sid = lax.axis_index("s")
    row0 = wid * _ROWS_PER_W

    def rows(ci):
        return pl.multiple_of(row0 + ci * _C, _C)

    def hbm_x(j):
        ci, b = _JOBS[j]
        return x_hbm.at[b, pl.ds(rows(ci), _C), :]

    def hbm_o(j):
        ci, b = _JOBS[j]
        return o_hbm.at[b, pl.ds(rows(ci), _C), :]

    def start_w(ci):
        pltpu.async_copy(w_hbm.at[pl.ds(rows(ci), _C), :],
                         wbufs[ci % 3], sws[ci % 3])

    def wait_w(ci):
        pltpu.make_async_copy(w_hbm.at[pl.ds(rows(ci), _C), :],
                              wbufs[ci % 3], sws[ci % 3]).wait()

    def start_in(j):
        pltpu.async_copy(hbm_x(j), tbufs[j % 3], s_in[j % 3])

    def wait_in(j):
        pltpu.make_async_copy(hbm_x(j), tbufs[j % 3], s_in[j % 3]).wait()

    def add(j):
        ci, b = _JOBS[j]
        xb = tbufs[j % 3]
        wb = wbufs[ci % 3]

        def row_body(r, carry):
            @plsc.parallel_loop(0, _D, step=_L, unroll=8)
            def _add(i):
                xb[r, pl.ds(i, _L)] = xb[r, pl.ds(i, _L)] + wb[r, pl.ds(i, _L)]

            return carry

        lax.fori_loop(0, _C, row_body, 0)

    def start_back(j):
        pltpu.async_copy(tbufs[j % 3], sp.at[sid, j % 2], s_back[j % 3])

    def wait_back(j):
        pltpu.make_async_copy(tbufs[j % 3], sp.at[sid, j % 2],
                              s_back[j % 3]).wait()

    def start_out(j):
        pltpu.async_copy(sp.at[sid, j % 2], hbm_o(j), s_out[j % 2])

    def wait_out(j):
        pltpu.make_async_copy(sp.at[sid, j % 2], hbm_o(j),
                              s_out[j % 2]).wait()

    start_w(0)
    start_w(1)

    for t in range(_NJ + 3):
        j3 = t - 3
        if 0 <= j3 < _NJ:
            wait_out(j3)
        j2 = t - 2
        if 0 <= j2 < _NJ:
            wait_back(j2)
            start_out(j2)
        j1 = t - 1
        if 0 <= j1 < _NJ:
            ci1, b1 = _JOBS[j1]
            if b1 == 0:
                wait_w(ci1)
            wait_in(j1)
            add(j1)
            start_back(j1)
            if b1 == _B - 1 and ci1 + 2 < _NCH:
                start_w(ci1 + 2)
        if t < _NJ:
            start_in(t)


def kernel(x, weight):
    mesh = plsc.VectorSubcoreMesh(core_axis_name="c", subcore_axis_name="s")
    out = pl.kernel(
        _sc_body,
        out_type=jax.ShapeDtypeStruct((_B, _S, _D), jnp.float32),
        mesh=mesh,
        scratch_types=[
            pltpu.VMEM((_C, _D), jnp.float32),
            pltpu.VMEM((_C, _D), jnp.float32),
            pltpu.VMEM((_C, _D), jnp.float32),
            pltpu.VMEM((_C, _D), jnp.float32),
            pltpu.VMEM((_C, _D), jnp.float32),
            pltpu.VMEM((_C, _D), jnp.float32),
            pltpu.VMEM_SHARED((_NS, 2, _C, _D), jnp.float32),
            pltpu.SemaphoreType.DMA,
            pltpu.SemaphoreType.DMA,
            pltpu.SemaphoreType.DMA,
            pltpu.SemaphoreType.DMA,
            pltpu.SemaphoreType.DMA,
            pltpu.SemaphoreType.DMA,
            pltpu.SemaphoreType.DMA,
            pltpu.SemaphoreType.DMA,
            pltpu.SemaphoreType.DMA,
            pltpu.SemaphoreType.DMA,
            pltpu.SemaphoreType.DMA,
        ],
    )(x, weight)
    return out


# speedup vs baseline: 1.2946x; 1.2946x over previous
import jax
import jax.numpy as jnp
from jax import lax
from jax.experimental import pallas as pl
from jax.experimental.pallas import tpu as pltpu
from jax.experimental.pallas import tpu_sc as plsc

_B, _S, _D = 4, 4096, 1024
_NC, _NS, _L = 2, 16, 16
_NW = _NC * _NS
_ROWS_PER_W = _S // _NW
_C = 32
_CW = 16
_NCH = _ROWS_PER_W // _C
_NCW = _ROWS_PER_W // _CW
_JOBS = [(ci, b) for ci in range(_NCH) for b in range(_B)]


def _sc_body(x_hbm, w_hbm, o_hbm,
             wb0, wb1, wb2, xb0, xb1,
             sw0, sw1, sw2, si0, si1, so0, so1):
    wbufs = (wb0, wb1, wb2)
    xbufs = (xb0, xb1)
    sws = (sw0, sw1, sw2)
    sis = (si0, si1)
    sos = (so0, so1)

    wid = lax.axis_index("s") * _NC + lax.axis_index("c")
    row0 = wid * _ROWS_PER_W

    def rows(ci):
        return pl.multiple_of(row0 + ci * _C, _C)

    def rows_w(wci):
        return pl.multiple_of(row0 + wci * _CW, _CW)

    def start_w(wci):
        pltpu.async_copy(w_hbm.at[pl.ds(rows_w(wci), _CW), :],
                         wbufs[wci % 3], sws[wci % 3])

    def wait_w(wci):
        pltpu.make_async_copy(w_hbm.at[pl.ds(rows_w(wci), _CW), :],
                              wbufs[wci % 3], sws[wci % 3]).wait()

    def start_in(j):
        ci, b = _JOBS[j]
        pltpu.async_copy(x_hbm.at[b, pl.ds(rows(ci), _C), :],
                         xbufs[j % 2], sis[j % 2])

    def wait_in(j):
        ci, b = _JOBS[j]
        pltpu.make_async_copy(x_hbm.at[b, pl.ds(rows(ci), _C), :],
                              xbufs[j % 2], sis[j % 2]).wait()

    def start_out(j):
        ci, b = _JOBS[j]
        pltpu.async_copy(xbufs[j % 2],
                         o_hbm.at[b, pl.ds(rows(ci), _C), :], sos[j % 2])

    def wait_out(j):
        ci, b = _JOBS[j]
        pltpu.make_async_copy(xbufs[j % 2],
                              o_hbm.at[b, pl.ds(rows(ci), _C), :],
                              sos[j % 2]).wait()

    start_w(0)
    start_w(1)
    start_in(0)

    for j, (ci, b) in enumerate(_JOBS):
        jn = j + 1
        if jn < len(_JOBS):
            if jn - 2 >= 0:
                wait_out(jn - 2)
            start_in(jn)
        if b == 0:
            wait_w(2 * ci)
            wait_w(2 * ci + 1)
        wait_in(j)

        xb = xbufs[j % 2]
        for half in range(2):
            wb = wbufs[(2 * ci + half) % 3]
            base = half * _CW

            def row_body(r, carry, xb=xb, wb=wb, base=base):
                @plsc.parallel_loop(0, _D, step=_L, unroll=8)
                def _add(i):
                    xb[base + r, pl.ds(i, _L)] = (
                        xb[base + r, pl.ds(i, _L)] + wb[r, pl.ds(i, _L)]
                    )

                return carry

            lax.fori_loop(0, _CW, row_body, 0)

        if b == 2 and ci + 1 < _NCH:
            start_w(2 * (ci + 1))
        if b == 3 and ci + 1 < _NCH:
            start_w(2 * (ci + 1) + 1)

        start_out(j)

    for j in range(len(_JOBS) - 2, len(_JOBS)):
        wait_out(j)


def kernel(x, weight):
    mesh = plsc.VectorSubcoreMesh(core_axis_name="c", subcore_axis_name="s")
    out = pl.kernel(
        _sc_body,
        out_type=jax.ShapeDtypeStruct((_B, _S, _D), jnp.float32),
        mesh=mesh,
        scratch_types=[
            pltpu.VMEM((_CW, _D), jnp.float32),
            pltpu.VMEM((_CW, _D), jnp.float32),
            pltpu.VMEM((_CW, _D), jnp.float32),
            pltpu.VMEM((_C, _D), jnp.float32),
            pltpu.VMEM((_C, _D), jnp.float32),
            pltpu.SemaphoreType.DMA,
            pltpu.SemaphoreType.DMA,
            pltpu.SemaphoreType.DMA,
            pltpu.SemaphoreType.DMA,
            pltpu.SemaphoreType.DMA,
            pltpu.SemaphoreType.DMA,
            pltpu.SemaphoreType.DMA,
        ],
    )(x, weight)
    return out


# speedup vs baseline: 1.3389x; 1.0342x over previous
import jax
import jax.numpy as jnp
from jax import lax
from jax.experimental import pallas as pl
from jax.experimental.pallas import tpu as pltpu
from jax.experimental.pallas import tpu_sc as plsc

_B, _S, _D = 4, 4096, 1024
_NC, _NS, _L = 2, 16, 16
_NW = _NC * _NS
_ROWS_PER_W = _S // _NW
_C = 16
_NCH = _ROWS_PER_W // _C
_JOBS = [(ci, b) for ci in range(_NCH) for b in range(_B)]


def _sc_body(x_hbm, w_hbm, o_hbm,
             wb0, wb1, xb0, xb1, xb2,
             sw0, sw1, si0, si1, si2, so0, so1, so2):
    wbufs = (wb0, wb1)
    xbufs = (xb0, xb1, xb2)
    sws = (sw0, sw1)
    sis = (si0, si1, si2)
    sos = (so0, so1, so2)

    wid = lax.axis_index("s") * _NC + lax.axis_index("c")
    row0 = wid * _ROWS_PER_W

    def rows(ci):
        return pl.multiple_of(row0 + ci * _C, _C)

    def start_w(ci):
        pltpu.async_copy(w_hbm.at[pl.ds(rows(ci), _C), :],
                         wbufs[ci % 2], sws[ci % 2])

    def wait_w(ci):
        pltpu.make_async_copy(w_hbm.at[pl.ds(rows(ci), _C), :],
                              wbufs[ci % 2], sws[ci % 2]).wait()

    def start_in(j):
        ci, b = _JOBS[j]
        pltpu.async_copy(x_hbm.at[b, pl.ds(rows(ci), _C), :],
                         xbufs[j % 3], sis[j % 3])

    def wait_in(j):
        ci, b = _JOBS[j]
        pltpu.make_async_copy(x_hbm.at[b, pl.ds(rows(ci), _C), :],
                              xbufs[j % 3], sis[j % 3]).wait()

    def start_out(j):
        ci, b = _JOBS[j]
        pltpu.async_copy(xbufs[j % 3],
                         o_hbm.at[b, pl.ds(rows(ci), _C), :], sos[j % 3])

    def wait_out(j):
        ci, b = _JOBS[j]
        pltpu.make_async_copy(xbufs[j % 3],
                              o_hbm.at[b, pl.ds(rows(ci), _C), :],
                              sos[j % 3]).wait()

    start_w(0)
    start_in(0)
    start_in(1)

    for j, (ci, b) in enumerate(_JOBS):
        if b == 0 and ci + 1 < _NCH:
            start_w(ci + 1)
        jn = j + 2
        if jn < len(_JOBS):
            if jn - 3 >= 0:
                wait_out(jn - 3)
            start_in(jn)
        if b == 0:
            wait_w(ci)
        wait_in(j)

        xb = xbufs[j % 3]
        wb = wbufs[ci % 2]

        def row_body(r, carry):
            @plsc.parallel_loop(0, _D, step=_L, unroll=8)
            def _add(i):
                xb[r, pl.ds(i, _L)] = xb[r, pl.ds(i, _L)] + wb[r, pl.ds(i, _L)]

            return carry

        lax.fori_loop(0, _C, row_body, 0)
        start_out(j)

    for j in range(len(_JOBS) - 3, len(_JOBS)):
        wait_out(j)


def kernel(x, weight):
    mesh = plsc.VectorSubcoreMesh(core_axis_name="c", subcore_axis_name="s")
    out = pl.kernel(
        _sc_body,
        out_type=jax.ShapeDtypeStruct((_B, _S, _D), jnp.float32),
        mesh=mesh,
        scratch_types=[
            pltpu.VMEM((_C, _D), jnp.float32),
            pltpu.VMEM((_C, _D), jnp.float32),
            pltpu.VMEM((_C, _D), jnp.float32),
            pltpu.VMEM((_C, _D), jnp.float32),
            pltpu.VMEM((_C, _D), jnp.float32),
            pltpu.SemaphoreType.DMA,
            pltpu.SemaphoreType.DMA,
            pltpu.SemaphoreType.DMA,
            pltpu.SemaphoreType.DMA,
            pltpu.SemaphoreType.DMA,
            pltpu.SemaphoreType.DMA,
            pltpu.SemaphoreType.DMA,
            pltpu.SemaphoreType.DMA,
        ],
    )(x, weight)
    return out
